# Initial kernel scaffold; baseline (speedup 1.0000x reference)
import jax
import jax.numpy as jnp
from jax.experimental import pallas as pl


def kernel(x, edge_index, W1_src, W1_dst, attn1, b1, W2_src, W2_dst, attn2, b2):
    raise NotImplementedError("write your pallas kernel here")


# speedup vs baseline: 25.8050x; 25.8050x over previous
import functools

import jax
import jax.numpy as jnp
from jax import lax
from jax.experimental import pallas as pl
from jax.experimental.pallas import tpu as pltpu
from jax.experimental.pallas import tpu_sc as plsc

N = 10000
E = 320000
D_IN = 128
H1, F1 = 8, 8
H2, F2 = 1, 40

NP = 10240
TILES = 32
CHUNK = 128
EPT = ((E + TILES * CHUNK - 1) // (TILES * CHUNK)) * CHUNK
EP = EPT * TILES
NCHUNK = EPT // CHUNK
BLK = 1024


def _make_sc_edge_kernel(width, n_heads):
  accw = 2 * width
  nq = width // 16
  rows_per_tile = NP // 16
  mesh = plsc.VectorSubcoreMesh(core_axis_name="c", subcore_axis_name="s",
                                num_cores=2, num_subcores=16)

  @functools.partial(
      pl.kernel,
      out_type=jax.ShapeDtypeStruct((2, NP, accw), jnp.float32),
      mesh=mesh,
      compiler_params=pltpu.CompilerParams(use_tc_tiling_on_sc=False),
      scratch_types=[
          pltpu.VMEM((CHUNK,), jnp.int32),
          pltpu.VMEM((CHUNK,), jnp.int32),
          pltpu.VMEM((CHUNK, width), jnp.float32),
          pltpu.VMEM((CHUNK, width), jnp.float32),
          pltpu.VMEM((CHUNK, accw), jnp.float32),
          pltpu.VMEM((width,), jnp.float32),
          pltpu.VMEM_SHARED((NP, accw), jnp.float32),
          pltpu.SemaphoreType.DMA,
          pltpu.SemaphoreType.DMA,
      ],
  )
  def edge_kernel(src_hbm, dst_hbm, fs_hbm, fd_hbm, attn_hbm, out_hbm,
                  idx_s, idx_d, rows_s, rows_d, contrib, attn_v, acc,
                  sem1, sem2):
    cid = lax.axis_index("c")
    sid = lax.axis_index("s")
    wid = sid * 2 + cid

    zeros16 = jnp.zeros((16,), jnp.float32)

    def zero_row(i, _):
      def zero_col(j, _):
        contrib[i, pl.ds(j * 16, 16)] = zeros16
        return 0
      return lax.fori_loop(0, accw // 16, zero_col, 0)
    lax.fori_loop(0, CHUNK, zero_row, 0)
    for r in range(rows_per_tile // CHUNK):
      pltpu.sync_copy(contrib,
                      acc.at[pl.ds(sid * rows_per_tile + r * CHUNK, CHUNK)])
    plsc.subcore_barrier()

    pltpu.sync_copy(attn_hbm, attn_v)
    attn_q = [attn_v[pl.ds(q * 16, 16)] for q in range(nq)]

    lanes = lax.iota(jnp.int32, 16)
    perms = [lanes ^ 1, lanes ^ 2, lanes ^ 4, lanes ^ 8]

    def bfly(v, p):
      return v + jnp.take_along_axis(v, p, axis=0,
                                     mode="promise_in_bounds")

    def chunk_body(c, _):
      base = (wid * NCHUNK + c) * CHUNK
      pltpu.sync_copy(src_hbm.at[pl.ds(base, CHUNK)], idx_s)
      pltpu.sync_copy(dst_hbm.at[pl.ds(base, CHUNK)], idx_d)
      cp1 = pltpu.async_copy(fs_hbm.at[idx_s], rows_s, sem1)
      cp2 = pltpu.async_copy(fd_hbm.at[idx_d], rows_d, sem2)
      cp1.wait()
      cp2.wait()

      if n_heads > 1:
        def edge_body(e, _):
          for q in range(nq):
            s_q = rows_s[e, pl.ds(q * 16, 16)]
            d_q = rows_d[e, pl.ds(q * 16, 16)]
            t = s_q + d_q
            lr = jnp.maximum(t, 0.2 * t)
            m = lr * attn_q[q]
            b = bfly(m, perms[0])
            b = bfly(b, perms[1])
            b = bfly(b, perms[2])
            p = jnp.exp(b)
            contrib[e, pl.ds(q * 16, 16)] = p * s_q
            contrib[e, pl.ds(width + q * 16, 16)] = p
          return 0
      else:
        def edge_body(e, _):
          s_qs = []
          tot = zeros16
          for q in range(nq):
            s_q = rows_s[e, pl.ds(q * 16, 16)]
            d_q = rows_d[e, pl.ds(q * 16, 16)]
            t = s_q + d_q
            lr = jnp.maximum(t, 0.2 * t)
            tot = tot + lr * attn_q[q]
            s_qs.append(s_q)
          for p_idx in perms:
            tot = bfly(tot, p_idx)
          p = jnp.exp(tot)
          for q in range(nq):
            contrib[e, pl.ds(q * 16, 16)] = p * s_qs[q]
            contrib[e, pl.ds(width + q * 16, 16)] = p
          return 0

      lax.fori_loop(0, CHUNK, edge_body, 0)
      pltpu.sync_copy(contrib, acc.at[idx_d], add=True)
      return 0

    lax.fori_loop(0, NCHUNK, chunk_body, 0)

    plsc.subcore_barrier()
    for r in range(rows_per_tile // CHUNK):
      off = sid * rows_per_tile + r * CHUNK
      pltpu.sync_copy(acc.at[pl.ds(off, CHUNK)],
                      out_hbm.at[cid, pl.ds(off, CHUNK)])

  return edge_kernel


def _mm1_body(x_ref, ws_ref, wd_ref, fs_ref, fd_ref):
  xb = x_ref[...]
  fs_ref[...] = jnp.dot(xb, ws_ref[...], preferred_element_type=jnp.float32)
  fd_ref[...] = jnp.dot(xb, wd_ref[...], preferred_element_type=jnp.float32)


def _mid_body(acc_ref, b1_ref, ws_ref, wd_ref, fs2_ref, fd2_ref):
  a = acc_ref[...]
  s = a[0] + a[1]
  num = s[:, :64]
  den = s[:, 64:]
  nz = den != 0.0
  h = jnp.where(nz, num / jnp.where(nz, den, 1.0), 0.0) + b1_ref[...]
  h = jnp.where(h > 0.0, h, jnp.exp(h) - 1.0)
  fs2_ref[...] = jnp.dot(h, ws_ref[...], preferred_element_type=jnp.float32)
  fd2_ref[...] = jnp.dot(h, wd_ref[...], preferred_element_type=jnp.float32)


def _fin_body(acc_ref, b2_ref, o_ref):
  a = acc_ref[...]
  s = a[0] + a[1]
  num = s[:, :48]
  den = s[:, 48:]
  nz = den != 0.0
  o_ref[...] = jnp.where(nz, num / jnp.where(nz, den, 1.0), 0.0) + b2_ref[...]


def kernel(x, edge_index, W1_src, W1_dst, attn1, b1, W2_src, W2_dst,
           attn2, b2):
  f32 = jnp.float32
  npad = NP - N
  x_p = jnp.pad(x, ((0, npad), (0, 0)))
  pad_e = jnp.full((EP - E,), N, jnp.int32)
  src = jnp.concatenate([edge_index[0], pad_e])
  dst = jnp.concatenate([edge_index[1], pad_e])

  attn1_flat = attn1.reshape(H1 * F1).astype(f32)
  attn2_flat = jnp.pad(attn2.reshape(H2 * F2), (0, 8)).astype(f32)
  W2s_p = jnp.pad(W2_src, ((0, 0), (0, 8)))
  W2d_p = jnp.pad(W2_dst, ((0, 0), (0, 8)))
  b1_2d = b1.reshape(1, 64)
  b2_2d = jnp.pad(b2, (0, 8)).reshape(1, 48)

  grid = (NP // BLK,)
  fs1, fd1 = pl.pallas_call(
      _mm1_body,
      grid=grid,
      in_specs=[
          pl.BlockSpec((BLK, D_IN), lambda i: (i, 0)),
          pl.BlockSpec((D_IN, 64), lambda i: (0, 0)),
          pl.BlockSpec((D_IN, 64), lambda i: (0, 0)),
      ],
      out_specs=[
          pl.BlockSpec((BLK, 64), lambda i: (i, 0)),
          pl.BlockSpec((BLK, 64), lambda i: (i, 0)),
      ],
      out_shape=[jax.ShapeDtypeStruct((NP, 64), f32)] * 2,
  )(x_p, W1_src, W1_dst)

  edge1 = _make_sc_edge_kernel(64, H1)
  acc1 = edge1(src, dst, fs1, fd1, attn1_flat)

  fs2, fd2 = pl.pallas_call(
      _mid_body,
      grid=grid,
      in_specs=[
          pl.BlockSpec((2, BLK, 128), lambda i: (0, i, 0)),
          pl.BlockSpec((1, 64), lambda i: (0, 0)),
          pl.BlockSpec((64, 48), lambda i: (0, 0)),
          pl.BlockSpec((64, 48), lambda i: (0, 0)),
      ],
      out_specs=[
          pl.BlockSpec((BLK, 48), lambda i: (i, 0)),
          pl.BlockSpec((BLK, 48), lambda i: (i, 0)),
      ],
      out_shape=[jax.ShapeDtypeStruct((NP, 48), f32)] * 2,
  )(acc1, b1_2d, W2s_p, W2d_p)

  edge2 = _make_sc_edge_kernel(48, H2)
  acc2 = edge2(src, dst, fs2, fd2, attn2_flat)

  out = pl.pallas_call(
      _fin_body,
      grid=grid,
      in_specs=[
          pl.BlockSpec((2, BLK, 96), lambda i: (0, i, 0)),
          pl.BlockSpec((1, 48), lambda i: (0, 0)),
      ],
      out_specs=pl.BlockSpec((BLK, 48), lambda i: (i, 0)),
      out_shape=jax.ShapeDtypeStruct((NP, 48), f32),
  )(acc2, b2_2d)

  return out[:N, :H2 * F2]


# speedup vs baseline: 31.4696x; 1.2195x over previous
import functools

import jax
import jax.numpy as jnp
from jax import lax
from jax.experimental import pallas as pl
from jax.experimental.pallas import tpu as pltpu
from jax.experimental.pallas import tpu_sc as plsc

N = 10000
E = 320000
D_IN = 128
H1, F1 = 8, 8
H2, F2 = 1, 40

NP = 10240
TILES = 32
CHUNK = 128
ACCN = 10048
RPT = ACCN // 16
RFULL = RPT // CHUNK
RTAIL = RPT - RFULL * CHUNK
ABLK = 1256
NCHUNK = (((E + TILES * CHUNK - 1) // (TILES * CHUNK)) + 1) // 2 * 2
EPT = NCHUNK * CHUNK
EP = EPT * TILES
BLK = 1024


def _make_sc_edge_kernel(width, n_heads):
  accw = 2 * width
  nq = width // 16
  mesh = plsc.VectorSubcoreMesh(core_axis_name="c", subcore_axis_name="s",
                                num_cores=2, num_subcores=16)

  @functools.partial(
      pl.kernel,
      out_type=jax.ShapeDtypeStruct((2, ACCN, accw), jnp.float32),
      mesh=mesh,
      compiler_params=pltpu.CompilerParams(use_tc_tiling_on_sc=False),
      scratch_types=[
          pltpu.VMEM((CHUNK,), jnp.int32),
          pltpu.VMEM((CHUNK,), jnp.int32),
          pltpu.VMEM((CHUNK,), jnp.int32),
          pltpu.VMEM((CHUNK,), jnp.int32),
          pltpu.VMEM((CHUNK, width), jnp.float32),
          pltpu.VMEM((CHUNK, width), jnp.float32),
          pltpu.VMEM((CHUNK, width), jnp.float32),
          pltpu.VMEM((CHUNK, width), jnp.float32),
          pltpu.VMEM((CHUNK, accw), jnp.float32),
          pltpu.VMEM((width,), jnp.float32),
          pltpu.VMEM_SHARED((ACCN, accw), jnp.float32),
          pltpu.SemaphoreType.DMA,
          pltpu.SemaphoreType.DMA,
          pltpu.SemaphoreType.DMA,
          pltpu.SemaphoreType.DMA,
      ],
  )
  def edge_kernel(src_hbm, dst_hbm, fs_hbm, fd_hbm, attn_hbm, out_hbm,
                  idx_s0, idx_s1, idx_d0, idx_d1, rows_s0, rows_s1,
                  rows_d0, rows_d1, contrib, attn_v, acc,
                  si0, si1, gg0, gg1):
    cid = lax.axis_index("c")
    sid = lax.axis_index("s")
    wid = sid * 2 + cid
    tile_base = wid * EPT

    idx_s = [idx_s0, idx_s1]
    idx_d = [idx_d0, idx_d1]
    rows_s = [rows_s0, rows_s1]
    rows_d = [rows_d0, rows_d1]
    si = [si0, si1]
    gg = [gg0, gg1]

    zeros16 = jnp.zeros((16,), jnp.float32)

    def zero_row(i, _):
      def zero_col(j, _):
        contrib[i, pl.ds(j * 16, 16)] = zeros16
        return 0
      return lax.fori_loop(0, accw // 16, zero_col, 0)
    lax.fori_loop(0, CHUNK, zero_row, 0)
    for r in range(RFULL):
      pltpu.sync_copy(contrib,
                      acc.at[pl.ds(sid * RPT + r * CHUNK, CHUNK)])
    pltpu.sync_copy(contrib.at[pl.ds(0, RTAIL)],
                    acc.at[pl.ds(sid * RPT + RFULL * CHUNK, RTAIL)])
    plsc.subcore_barrier()

    pltpu.sync_copy(attn_hbm, attn_v)
    attn_q = [attn_v[pl.ds(q * 16, 16)] for q in range(nq)]

    lanes = lax.iota(jnp.int32, 16)
    perms = [lanes ^ 1, lanes ^ 2, lanes ^ 4, lanes ^ 8]

    def bfly(v, p):
      return v + jnp.take_along_axis(v, p, axis=0,
                                     mode="promise_in_bounds")

    def chunk_base(c):
      return tile_base + jnp.minimum(c, NCHUNK - 1) * CHUNK

    def issue_idx(c, p):
      base = chunk_base(c)
      pltpu.async_copy(src_hbm.at[pl.ds(base, CHUNK)], idx_s[p], si[p])
      pltpu.async_copy(dst_hbm.at[pl.ds(base, CHUNK)], idx_d[p], si[p])

    def wait_idx(c, p):
      base = chunk_base(c)
      pltpu.make_async_copy(src_hbm.at[pl.ds(base, CHUNK)], idx_s[p],
                            si[p]).wait()
      pltpu.make_async_copy(dst_hbm.at[pl.ds(base, CHUNK)], idx_d[p],
                            si[p]).wait()

    def issue_gather(p):
      pltpu.async_copy(fs_hbm.at[idx_s[p]], rows_s[p], gg[p])
      pltpu.async_copy(fd_hbm.at[idx_d[p]], rows_d[p], gg[p])

    def wait_gather(p):
      pltpu.make_async_copy(fs_hbm.at[idx_s[p]], rows_s[p], gg[p]).wait()
      pltpu.make_async_copy(fd_hbm.at[idx_d[p]], rows_d[p], gg[p]).wait()

    if n_heads > 1:
      def make_edge_body(p):
        def edge_body(e, _):
          for q in range(nq):
            s_q = rows_s[p][e, pl.ds(q * 16, 16)]
            d_q = rows_d[p][e, pl.ds(q * 16, 16)]
            t = s_q + d_q
            lr = jnp.maximum(t, 0.2 * t)
            m = lr * attn_q[q]
            b = bfly(m, perms[0])
            b = bfly(b, perms[1])
            b = bfly(b, perms[2])
            pv = jnp.exp(b)
            contrib[e, pl.ds(q * 16, 16)] = pv * s_q
            contrib[e, pl.ds(width + q * 16, 16)] = pv
          return 0
        return edge_body
    else:
      def make_edge_body(p):
        def edge_body(e, _):
          s_qs = []
          tot = zeros16
          for q in range(nq):
            s_q = rows_s[p][e, pl.ds(q * 16, 16)]
            d_q = rows_d[p][e, pl.ds(q * 16, 16)]
            t = s_q + d_q
            lr = jnp.maximum(t, 0.2 * t)
            tot = tot + lr * attn_q[q]
            s_qs.append(s_q)
          for p_idx in perms:
            tot = bfly(tot, p_idx)
          pv = jnp.exp(tot)
          for q in range(nq):
            contrib[e, pl.ds(q * 16, 16)] = pv * s_qs[q]
            contrib[e, pl.ds(width + q * 16, 16)] = pv
          return 0
        return edge_body

    edge_bodies = [make_edge_body(0), make_edge_body(1)]

    def sub_iter(c, p):
      q = 1 - p
      wait_idx(c + 1, q)
      issue_gather(q)
      wait_gather(p)
      lax.fori_loop(0, CHUNK, edge_bodies[p], 0, unroll=4)
      pltpu.sync_copy(contrib, acc.at[idx_d[p]], add=True)
      issue_idx(c + 2, p)

    issue_idx(0, 0)
    wait_idx(0, 0)
    issue_gather(0)
    issue_idx(1, 1)

    def group_body(g, _):
      sub_iter(2 * g, 0)
      sub_iter(2 * g + 1, 1)
      return 0
    lax.fori_loop(0, NCHUNK // 2, group_body, 0)

    wait_gather(0)
    wait_idx(NCHUNK + 1, 1)

    plsc.subcore_barrier()
    for r in range(RFULL):
      off = sid * RPT + r * CHUNK
      pltpu.sync_copy(acc.at[pl.ds(off, CHUNK)],
                      out_hbm.at[cid, pl.ds(off, CHUNK)])
    off = sid * RPT + RFULL * CHUNK
    pltpu.sync_copy(acc.at[pl.ds(off, RTAIL)],
                    out_hbm.at[cid, pl.ds(off, RTAIL)])

  return edge_kernel


def _mm1_body(x_ref, ws_ref, wd_ref, fs_ref, fd_ref):
  xb = x_ref[...]
  fs_ref[...] = jnp.dot(xb, ws_ref[...], preferred_element_type=jnp.float32)
  fd_ref[...] = jnp.dot(xb, wd_ref[...], preferred_element_type=jnp.float32)


def _mid_body(acc_ref, b1_ref, ws_ref, wd_ref, fs2_ref, fd2_ref):
  a = acc_ref[...]
  s = a[0] + a[1]
  num = s[:, :64]
  den = s[:, 64:]
  nz = den != 0.0
  h = jnp.where(nz, num / jnp.where(nz, den, 1.0), 0.0) + b1_ref[...]
  h = jnp.where(h > 0.0, h, jnp.exp(h) - 1.0)
  fs2_ref[...] = jnp.dot(h, ws_ref[...], preferred_element_type=jnp.float32)
  fd2_ref[...] = jnp.dot(h, wd_ref[...], preferred_element_type=jnp.float32)


def _fin_body(acc_ref, b2_ref, o_ref):
  a = acc_ref[...]
  s = a[0] + a[1]
  num = s[:, :48]
  den = s[:, 48:]
  nz = den != 0.0
  o_ref[...] = jnp.where(nz, num / jnp.where(nz, den, 1.0), 0.0) + b2_ref[...]


def kernel(x, edge_index, W1_src, W1_dst, attn1, b1, W2_src, W2_dst,
           attn2, b2):
  f32 = jnp.float32
  npad = NP - N
  x_p = jnp.pad(x, ((0, npad), (0, 0)))
  pad_e = jnp.full((EP - E,), N, jnp.int32)
  src = jnp.concatenate([edge_index[0], pad_e])
  dst = jnp.concatenate([edge_index[1], pad_e])

  attn1_flat = attn1.reshape(H1 * F1).astype(f32)
  attn2_flat = jnp.pad(attn2.reshape(H2 * F2), (0, 8)).astype(f32)
  W2s_p = jnp.pad(W2_src, ((0, 0), (0, 8)))
  W2d_p = jnp.pad(W2_dst, ((0, 0), (0, 8)))
  b1_2d = b1.reshape(1, 64)
  b2_2d = jnp.pad(b2, (0, 8)).reshape(1, 48)

  grid = (NP // BLK,)
  fs1, fd1 = pl.pallas_call(
      _mm1_body,
      grid=grid,
      in_specs=[
          pl.BlockSpec((BLK, D_IN), lambda i: (i, 0)),
          pl.BlockSpec((D_IN, 64), lambda i: (0, 0)),
          pl.BlockSpec((D_IN, 64), lambda i: (0, 0)),
      ],
      out_specs=[
          pl.BlockSpec((BLK, 64), lambda i: (i, 0)),
          pl.BlockSpec((BLK, 64), lambda i: (i, 0)),
      ],
      out_shape=[jax.ShapeDtypeStruct((NP, 64), f32)] * 2,
  )(x_p, W1_src, W1_dst)

  edge1 = _make_sc_edge_kernel(64, H1)
  acc1 = edge1(src, dst, fs1, fd1, attn1_flat)

  agrid = (ACCN // ABLK,)
  fs2, fd2 = pl.pallas_call(
      _mid_body,
      grid=agrid,
      in_specs=[
          pl.BlockSpec((2, ABLK, 128), lambda i: (0, i, 0)),
          pl.BlockSpec((1, 64), lambda i: (0, 0)),
          pl.BlockSpec((64, 48), lambda i: (0, 0)),
          pl.BlockSpec((64, 48), lambda i: (0, 0)),
      ],
      out_specs=[
          pl.BlockSpec((ABLK, 48), lambda i: (i, 0)),
          pl.BlockSpec((ABLK, 48), lambda i: (i, 0)),
      ],
      out_shape=[jax.ShapeDtypeStruct((ACCN, 48), f32)] * 2,
  )(acc1, b1_2d, W2s_p, W2d_p)

  edge2 = _make_sc_edge_kernel(48, H2)
  acc2 = edge2(src, dst, fs2, fd2, attn2_flat)

  out = pl.pallas_call(
      _fin_body,
      grid=agrid,
      in_specs=[
          pl.BlockSpec((2, ABLK, 96), lambda i: (0, i, 0)),
          pl.BlockSpec((1, 48), lambda i: (0, 0)),
      ],
      out_specs=pl.BlockSpec((ABLK, 48), lambda i: (i, 0)),
      out_shape=jax.ShapeDtypeStruct((ACCN, 48), f32),
  )(acc2, b2_2d)

  return out[:N, :H2 * F2]


# speedup vs baseline: 32.8050x; 1.0424x over previous
import functools

import jax
import jax.numpy as jnp
from jax import lax
from jax.experimental import pallas as pl
from jax.experimental.pallas import tpu as pltpu
from jax.experimental.pallas import tpu_sc as plsc

N = 10000
E = 320000
D_IN = 128
H1, F1 = 8, 8
H2, F2 = 1, 40

NP = 10240
TILES = 32
CHUNK = 128
ACCN = 10048
RPT = ACCN // 16
RFULL = RPT // CHUNK
RTAIL = RPT - RFULL * CHUNK
ABLK = 1256
NCHUNK = (((E + TILES * CHUNK - 1) // (TILES * CHUNK)) + 3) // 4 * 4
EPT = NCHUNK * CHUNK
EP = EPT * TILES
BLK = 1024


def _make_sc_edge_kernel(width, n_heads):
  accw = width + 16
  nq = width // 16
  mesh = plsc.VectorSubcoreMesh(core_axis_name="c", subcore_axis_name="s",
                                num_cores=2, num_subcores=16)

  @functools.partial(
      pl.kernel,
      out_type=jax.ShapeDtypeStruct((2, ACCN, accw), jnp.float32),
      mesh=mesh,
      compiler_params=pltpu.CompilerParams(use_tc_tiling_on_sc=False),
      scratch_types=[
          pltpu.VMEM((CHUNK,), jnp.int32),
          pltpu.VMEM((CHUNK,), jnp.int32),
          pltpu.VMEM((CHUNK,), jnp.int32),
          pltpu.VMEM((CHUNK,), jnp.int32),
          pltpu.VMEM((CHUNK,), jnp.int32),
          pltpu.VMEM((CHUNK,), jnp.int32),
          pltpu.VMEM((CHUNK,), jnp.int32),
          pltpu.VMEM((CHUNK,), jnp.int32),
          pltpu.VMEM((CHUNK, width), jnp.float32),
          pltpu.VMEM((CHUNK, width), jnp.float32),
          pltpu.VMEM((CHUNK, width), jnp.float32),
          pltpu.VMEM((CHUNK, width), jnp.float32),
          pltpu.VMEM((CHUNK, accw), jnp.float32),
          pltpu.VMEM((CHUNK, accw), jnp.float32),
          pltpu.VMEM((width,), jnp.float32),
          pltpu.VMEM_SHARED((ACCN, accw), jnp.float32),
          pltpu.SemaphoreType.DMA,
          pltpu.SemaphoreType.DMA,
          pltpu.SemaphoreType.DMA,
          pltpu.SemaphoreType.DMA,
          pltpu.SemaphoreType.DMA,
          pltpu.SemaphoreType.DMA,
          pltpu.SemaphoreType.DMA,
          pltpu.SemaphoreType.DMA,
      ],
  )
  def edge_kernel(src_hbm, dst_hbm, fs_hbm, fd_hbm, attn_hbm, out_hbm,
                  idx_s0, idx_s1, idx_s2, idx_s3,
                  idx_d0, idx_d1, idx_d2, idx_d3,
                  rows_s0, rows_s1, rows_d0, rows_d1,
                  contrib0, contrib1, attn_v, acc,
                  ii0, ii1, ii2, ii3, gg0, gg1, ss0, ss1):
    cid = lax.axis_index("c")
    sid = lax.axis_index("s")
    wid = sid * 2 + cid
    tile_base = wid * EPT

    idx_s = [idx_s0, idx_s1, idx_s2, idx_s3]
    idx_d = [idx_d0, idx_d1, idx_d2, idx_d3]
    rows_s = [rows_s0, rows_s1]
    rows_d = [rows_d0, rows_d1]
    contrib = [contrib0, contrib1]
    ii = [ii0, ii1, ii2, ii3]
    gg = [gg0, gg1]
    ss = [ss0, ss1]

    zeros16 = jnp.zeros((16,), jnp.float32)

    def zero_row(i, _):
      def zero_col(j, _):
        contrib0[i, pl.ds(j * 16, 16)] = zeros16
        return 0
      return lax.fori_loop(0, accw // 16, zero_col, 0)
    lax.fori_loop(0, CHUNK, zero_row, 0)
    for r in range(RFULL):
      pltpu.sync_copy(contrib0,
                      acc.at[pl.ds(sid * RPT + r * CHUNK, CHUNK)])
    pltpu.sync_copy(contrib0.at[pl.ds(0, RTAIL)],
                    acc.at[pl.ds(sid * RPT + RFULL * CHUNK, RTAIL)])
    plsc.subcore_barrier()

    pltpu.sync_copy(attn_hbm, attn_v)
    attn_q = [attn_v[pl.ds(q * 16, 16)] for q in range(nq)]

    lanes = lax.iota(jnp.int32, 16)
    perms = [lanes ^ 1, lanes ^ 2, lanes ^ 4, lanes ^ 8]
    pcomp = [jnp.where(lanes == 2 * q + 1, 8, 0) for q in range(4)]
    mask_a = lanes < 2
    mask_b = lanes < 6
    mask_c = lanes < 4

    def bfly(v, p):
      return v + jnp.take_along_axis(v, p, axis=0,
                                     mode="promise_in_bounds")

    def chunk_base(c):
      return tile_base + jnp.minimum(c, NCHUNK - 1) * CHUNK

    def issue_idx(c, j):
      base = chunk_base(c)
      pltpu.async_copy(src_hbm.at[pl.ds(base, CHUNK)], idx_s[j], ii[j])
      pltpu.async_copy(dst_hbm.at[pl.ds(base, CHUNK)], idx_d[j], ii[j])

    def wait_idx(c, j):
      base = chunk_base(c)
      pltpu.make_async_copy(src_hbm.at[pl.ds(base, CHUNK)], idx_s[j],
                            ii[j]).wait()
      pltpu.make_async_copy(dst_hbm.at[pl.ds(base, CHUNK)], idx_d[j],
                            ii[j]).wait()

    def issue_gather(j, p):
      pltpu.async_copy(fs_hbm.at[idx_s[j]], rows_s[p], gg[p])
      pltpu.async_copy(fd_hbm.at[idx_d[j]], rows_d[p], gg[p])

    def wait_gather(j, p):
      pltpu.make_async_copy(fs_hbm.at[idx_s[j]], rows_s[p], gg[p]).wait()
      pltpu.make_async_copy(fd_hbm.at[idx_d[j]], rows_d[p], gg[p]).wait()

    def issue_scatter(j, p):
      pltpu.async_copy(contrib[p], acc.at[idx_d[j]], ss[p], add=True)

    def wait_scatter(j, p):
      pltpu.make_async_copy(contrib[p], acc.at[idx_d[j]], ss[p]).wait()

    if n_heads > 1:
      def make_edge_body(p):
        def edge_body(e, _):
          pvs = []
          for q in range(nq):
            s_q = rows_s[p][e, pl.ds(q * 16, 16)]
            d_q = rows_d[p][e, pl.ds(q * 16, 16)]
            t = s_q + d_q
            lr = jnp.maximum(t, 0.2 * t)
            m = lr * attn_q[q]
            b = bfly(m, perms[0])
            b = bfly(b, perms[1])
            b = bfly(b, perms[2])
            pv = jnp.exp(b)
            contrib[p][e, pl.ds(q * 16, 16)] = pv * s_q
            pvs.append(pv)
          ts = [jnp.take_along_axis(pvs[q], pcomp[q], axis=0,
                                    mode="promise_in_bounds")
                for q in range(4)]
          w0 = jnp.where(mask_a, ts[0], ts[1])
          w1 = jnp.where(mask_b, ts[2], ts[3])
          contrib[p][e, pl.ds(width, 16)] = jnp.where(mask_c, w0, w1)
          return 0
        return edge_body
    else:
      def make_edge_body(p):
        def edge_body(e, _):
          s_qs = []
          tot = zeros16
          for q in range(nq):
            s_q = rows_s[p][e, pl.ds(q * 16, 16)]
            d_q = rows_d[p][e, pl.ds(q * 16, 16)]
            t = s_q + d_q
            lr = jnp.maximum(t, 0.2 * t)
            tot = tot + lr * attn_q[q]
            s_qs.append(s_q)
          for p_idx in perms:
            tot = bfly(tot, p_idx)
          pv = jnp.exp(tot)
          for q in range(nq):
            contrib[p][e, pl.ds(q * 16, 16)] = pv * s_qs[q]
          contrib[p][e, pl.ds(width, 16)] = pv
          return 0
        return edge_body

    edge_bodies = [make_edge_body(0), make_edge_body(1)]

    def sub_iter(c, k, steady):
      p = k % 2
      q = 1 - p
      kn = (k + 1) % 4
      kf = (k + 2) % 4
      wait_idx(c + 1, kn)
      issue_gather(kn, q)
      wait_gather(k, p)
      if steady:
        wait_scatter(kf, p)
      lax.fori_loop(0, CHUNK, edge_bodies[p], 0, unroll=8)
      issue_scatter(k, p)
      issue_idx(c + 2, kf)

    issue_idx(0, 0)
    wait_idx(0, 0)
    issue_gather(0, 0)
    issue_idx(1, 1)

    sub_iter(0, 0, False)
    sub_iter(1, 1, False)
    sub_iter(2, 2, True)
    sub_iter(3, 3, True)

    def group_body(g, _):
      c0 = 4 * g
      sub_iter(c0, 0, True)
      sub_iter(c0 + 1, 1, True)
      sub_iter(c0 + 2, 2, True)
      sub_iter(c0 + 3, 3, True)
      return 0
    lax.fori_loop(1, NCHUNK // 4, group_body, 0)

    wait_gather(0, 0)
    wait_idx(NCHUNK + 1, 1)
    wait_scatter(2, 0)
    wait_scatter(3, 1)

    plsc.subcore_barrier()
    for r in range(RFULL):
      off = sid * RPT + r * CHUNK
      pltpu.sync_copy(acc.at[pl.ds(off, CHUNK)],
                      out_hbm.at[cid, pl.ds(off, CHUNK)])
    off = sid * RPT + RFULL * CHUNK
    pltpu.sync_copy(acc.at[pl.ds(off, RTAIL)],
                    out_hbm.at[cid, pl.ds(off, RTAIL)])

  return edge_kernel


def _mm1_body(x_ref, ws_ref, wd_ref, fs_ref, fd_ref):
  xb = x_ref[...]
  fs_ref[...] = jnp.dot(xb, ws_ref[...], preferred_element_type=jnp.float32)
  fd_ref[...] = jnp.dot(xb, wd_ref[...], preferred_element_type=jnp.float32)


def _mid_body(acc_ref, b1_ref, ws_ref, wd_ref, fs2_ref, fd2_ref):
  a = acc_ref[...]
  s = a[0] + a[1]
  num = s[:, :64]
  den = jnp.repeat(s[:, 64:72], 8, axis=1)
  nz = den != 0.0
  h = jnp.where(nz, num / jnp.where(nz, den, 1.0), 0.0) + b1_ref[...]
  h = jnp.where(h > 0.0, h, jnp.exp(h) - 1.0)
  fs2_ref[...] = jnp.dot(h, ws_ref[...], preferred_element_type=jnp.float32)
  fd2_ref[...] = jnp.dot(h, wd_ref[...], preferred_element_type=jnp.float32)


def _fin_body(acc_ref, b2_ref, o_ref):
  a = acc_ref[...]
  s = a[0] + a[1]
  num = s[:, :48]
  den = s[:, 48:49]
  nz = den != 0.0
  o_ref[...] = jnp.where(nz, num / jnp.where(nz, den, 1.0), 0.0) + b2_ref[...]


def kernel(x, edge_index, W1_src, W1_dst, attn1, b1, W2_src, W2_dst,
           attn2, b2):
  f32 = jnp.float32
  npad = NP - N
  x_p = jnp.pad(x, ((0, npad), (0, 0)))
  pad_e = jnp.full((EP - E,), N, jnp.int32)
  src = jnp.concatenate([edge_index[0], pad_e])
  dst = jnp.concatenate([edge_index[1], pad_e])

  attn1_flat = attn1.reshape(H1 * F1).astype(f32)
  attn2_flat = jnp.pad(attn2.reshape(H2 * F2), (0, 8)).astype(f32)
  W2s_p = jnp.pad(W2_src, ((0, 0), (0, 8)))
  W2d_p = jnp.pad(W2_dst, ((0, 0), (0, 8)))
  b1_2d = b1.reshape(1, 64)
  b2_2d = jnp.pad(b2, (0, 8)).reshape(1, 48)

  grid = (NP // BLK,)
  fs1, fd1 = pl.pallas_call(
      _mm1_body,
      grid=grid,
      in_specs=[
          pl.BlockSpec((BLK, D_IN), lambda i: (i, 0)),
          pl.BlockSpec((D_IN, 64), lambda i: (0, 0)),
          pl.BlockSpec((D_IN, 64), lambda i: (0, 0)),
      ],
      out_specs=[
          pl.BlockSpec((BLK, 64), lambda i: (i, 0)),
          pl.BlockSpec((BLK, 64), lambda i: (i, 0)),
      ],
      out_shape=[jax.ShapeDtypeStruct((NP, 64), f32)] * 2,
  )(x_p, W1_src, W1_dst)

  edge1 = _make_sc_edge_kernel(64, H1)
  acc1 = edge1(src, dst, fs1, fd1, attn1_flat)

  agrid = (ACCN // ABLK,)
  fs2, fd2 = pl.pallas_call(
      _mid_body,
      grid=agrid,
      in_specs=[
          pl.BlockSpec((2, ABLK, 80), lambda i: (0, i, 0)),
          pl.BlockSpec((1, 64), lambda i: (0, 0)),
          pl.BlockSpec((64, 48), lambda i: (0, 0)),
          pl.BlockSpec((64, 48), lambda i: (0, 0)),
      ],
      out_specs=[
          pl.BlockSpec((ABLK, 48), lambda i: (i, 0)),
          pl.BlockSpec((ABLK, 48), lambda i: (i, 0)),
      ],
      out_shape=[jax.ShapeDtypeStruct((ACCN, 48), f32)] * 2,
  )(acc1, b1_2d, W2s_p, W2d_p)

  edge2 = _make_sc_edge_kernel(48, H2)
  acc2 = edge2(src, dst, fs2, fd2, attn2_flat)

  out = pl.pallas_call(
      _fin_body,
      grid=agrid,
      in_specs=[
          pl.BlockSpec((2, ABLK, 64), lambda i: (0, i, 0)),
          pl.BlockSpec((1, 48), lambda i: (0, 0)),
      ],
      out_specs=pl.BlockSpec((ABLK, 48), lambda i: (i, 0)),
      out_shape=jax.ShapeDtypeStruct((ACCN, 48), f32),
  )(acc2, b2_2d)

  return out[:N, :H2 * F2]
